# baseline (speedup 1.0000x reference)
import jax
import jax.numpy as jnp
from jax.experimental import pallas as pl

EMB_DIM = 128
BLOCK_ROWS = 4000


def _bcast_body(table_ref, out_ref):
    out_ref[...] = jnp.broadcast_to(table_ref[...], out_ref.shape)


def kernel(edge_index, table):
    n_edges = edge_index.shape[1]
    grid = (n_edges // BLOCK_ROWS,)
    return pl.pallas_call(
        _bcast_body,
        grid=grid,
        in_specs=[pl.BlockSpec((1, EMB_DIM), lambda i: (0, 0))],
        out_specs=pl.BlockSpec((BLOCK_ROWS, EMB_DIM), lambda i: (i, 0)),
        out_shape=jax.ShapeDtypeStruct((n_edges, EMB_DIM), jnp.float32),
    )(table)


# speedup vs baseline: 1.1830x; 1.1830x over previous
import jax
import jax.numpy as jnp
from jax.experimental import pallas as pl
from jax.experimental.pallas import tpu as pltpu

EMB_DIM = 128
BLOCK_ROWS = 2000


def _body(table_ref, out_ref, scratch, sem):
    scratch[...] = jnp.broadcast_to(table_ref[...], scratch.shape)
    n_chunks = out_ref.shape[0] // BLOCK_ROWS
    copies = [
        pltpu.make_async_copy(
            scratch, out_ref.at[pl.ds(j * BLOCK_ROWS, BLOCK_ROWS)], sem
        )
        for j in range(n_chunks)
    ]
    for c in copies:
        c.start()
    for c in copies:
        c.wait()


def kernel(edge_index, table):
    n_edges = edge_index.shape[1]
    return pl.pallas_call(
        _body,
        in_specs=[pl.BlockSpec((1, EMB_DIM), lambda: (0, 0))],
        out_specs=pl.BlockSpec(memory_space=pltpu.MemorySpace.HBM),
        out_shape=jax.ShapeDtypeStruct((n_edges, EMB_DIM), jnp.float32),
        scratch_shapes=[
            pltpu.VMEM((BLOCK_ROWS, EMB_DIM), jnp.float32),
            pltpu.SemaphoreType.DMA,
        ],
    )(table)
